# baseline (speedup 1.0000x reference)
import functools

import jax
import jax.numpy as jnp
from jax import lax
from jax.experimental import pallas as pl
from jax.experimental.pallas import tpu as pltpu
from jax.experimental.pallas import tpu_sc as plsc

HID = 64
NUM_SC_CORES = 2
NUM_SC_SUBCORES = 16
NUM_WORKERS = NUM_SC_CORES * NUM_SC_SUBCORES


def _sc_gather_rows(table, idx):
    n = idx.shape[0]
    d = table.shape[1]
    bpw = n // NUM_WORKERS
    mesh = plsc.VectorSubcoreMesh(core_axis_name="c", subcore_axis_name="s")

    @functools.partial(
        pl.kernel,
        mesh=mesh,
        out_type=jax.ShapeDtypeStruct((n, d), table.dtype),
        compiler_params=pltpu.CompilerParams(use_tc_tiling_on_sc=False),
        scratch_types=[
            pltpu.VMEM((bpw,), jnp.int32),
            pltpu.VMEM((bpw, d), table.dtype),
            pltpu.SemaphoreType.DMA,
        ],
    )
    def gather_kernel(table_hbm, idx_hbm, out_hbm, idx_v, rows_v, sem):
        wid = lax.axis_index("s") * NUM_SC_CORES + lax.axis_index("c")
        base = wid * bpw
        pltpu.sync_copy(idx_hbm.at[pl.ds(base, bpw)], idx_v)
        pltpu.async_copy(table_hbm.at[idx_v], rows_v, sem).wait()
        pltpu.sync_copy(rows_v, out_hbm.at[pl.ds(base, bpw)])

    return gather_kernel(table, idx)


def _gru_body(e_ref, h_ref, wir, wiz, win, whr, whz, whn, br, bz, bin_, bhn,
              o_ref):
    ev = e_ref[...]
    hv = h_ref[...]
    f32 = jnp.float32
    r = jax.nn.sigmoid(
        jnp.dot(ev, wir[...], preferred_element_type=f32)
        + jnp.dot(hv, whr[...], preferred_element_type=f32) + br[...])
    z = jax.nn.sigmoid(
        jnp.dot(ev, wiz[...], preferred_element_type=f32)
        + jnp.dot(hv, whz[...], preferred_element_type=f32) + bz[...])
    n = jnp.tanh(
        jnp.dot(ev, win[...], preferred_element_type=f32) + bin_[...]
        + r * (jnp.dot(hv, whn[...], preferred_element_type=f32) + bhn[...]))
    o_ref[...] = ((1.0 - z) * n + z * hv).astype(jnp.bfloat16)


def _score_body(xh_ref, ot_ref, b_ref, r_ref):
    acc = jnp.dot(xh_ref[...], ot_ref[...],
                  preferred_element_type=jnp.float32)
    r_ref[...] = acc + b_ref[...]


def kernel(X, H, Y, Wy, By, weight_ih, weight_hh, bias_ih, bias_hh):
    batch = X.shape[0]
    ny = Y.shape[0]
    X = X.astype(jnp.int32)
    Y = Y.astype(jnp.int32)

    E = _sc_gather_rows(Wy, X)
    O = _sc_gather_rows(Wy, Y)
    b = jnp.take(By, Y, axis=0).reshape(1, ny)
    Ot = O.T.astype(jnp.bfloat16)

    h0 = H[0]
    wir = weight_ih[0 * HID:1 * HID].T
    wiz = weight_ih[1 * HID:2 * HID].T
    win = weight_ih[2 * HID:3 * HID].T
    whr = weight_hh[0 * HID:1 * HID].T
    whz = weight_hh[1 * HID:2 * HID].T
    whn = weight_hh[2 * HID:3 * HID].T
    br = (bias_ih[0 * HID:1 * HID] + bias_hh[0 * HID:1 * HID]).reshape(1, HID)
    bz = (bias_ih[1 * HID:2 * HID] + bias_hh[1 * HID:2 * HID]).reshape(1, HID)
    bin_ = bias_ih[2 * HID:3 * HID].reshape(1, HID)
    bhn = bias_hh[2 * HID:3 * HID].reshape(1, HID)

    Xh = pl.pallas_call(
        _gru_body,
        out_shape=jax.ShapeDtypeStruct((batch, HID), jnp.bfloat16),
    )(E, h0, wir, wiz, win, whr, whz, whn, br, bz, bin_, bhn)

    bi = 512
    R = pl.pallas_call(
        _score_body,
        grid=(batch // bi,),
        in_specs=[
            pl.BlockSpec((bi, HID), lambda i: (i, 0)),
            pl.BlockSpec((HID, ny), lambda i: (0, 0)),
            pl.BlockSpec((1, ny), lambda i: (0, 0)),
        ],
        out_specs=pl.BlockSpec((bi, ny), lambda i: (i, 0)),
        out_shape=jax.ShapeDtypeStruct((batch, ny), jnp.float32),
        compiler_params=pltpu.CompilerParams(
            dimension_semantics=("arbitrary",)),
    )(Xh, Ot, b)
    return R


# speedup vs baseline: 1.6297x; 1.6297x over previous
import dataclasses
import functools

import jax
import jax.numpy as jnp
from jax import lax
from jax.experimental import pallas as pl
from jax.experimental.pallas import tpu as pltpu
from jax.experimental.pallas import tpu_sc as plsc

HID = 64
NUM_SC_CORES = 2
NUM_SC_SUBCORES = 16
NUM_WORKERS = NUM_SC_CORES * NUM_SC_SUBCORES
DIMS_PER_WORKER = HID // NUM_WORKERS


def _sc_gather_cols(wyt, x, y, byf):
    v = wyt.shape[1]
    nx = x.shape[0]
    ny = y.shape[0]
    n = nx + ny
    mesh = plsc.VectorSubcoreMesh(core_axis_name="c", subcore_axis_name="s")
    cp = pltpu.CompilerParams()
    if "needs_layout_passes" in pltpu.CompilerParams.__dataclass_fields__:
        cp = dataclasses.replace(cp, needs_layout_passes=False)

    @functools.partial(
        pl.kernel,
        mesh=mesh,
        compiler_params=cp,
        out_type=(
            jax.ShapeDtypeStruct((HID, nx), jnp.float32),
            jax.ShapeDtypeStruct((HID, ny), jnp.float32),
            jax.ShapeDtypeStruct((ny,), jnp.float32),
        ),
        scratch_types=[
            pltpu.VMEM((v,), jnp.float32),
            pltpu.VMEM((n,), jnp.int32),
            pltpu.VMEM((n,), jnp.float32),
            pltpu.VMEM((ny,), jnp.float32),
            pltpu.SemaphoreType.DMA,
        ],
    )
    def gather_kernel(wyt_hbm, x_hbm, y_hbm, byf_hbm, et_hbm, ot_hbm, b_hbm,
                      col_v, idx_v, row_v, b_v, sem):
        wid = lax.axis_index("s") * NUM_SC_CORES + lax.axis_index("c")
        pltpu.sync_copy(x_hbm, idx_v.at[pl.ds(0, nx)])
        pltpu.sync_copy(y_hbm, idx_v.at[pl.ds(nx, ny)])

        @pl.when(wid == NUM_WORKERS - 1)
        def _():
            pltpu.async_copy(byf_hbm.at[idx_v.at[pl.ds(nx, ny)]], b_v,
                             sem).wait()
            pltpu.sync_copy(b_v, b_hbm)

        @pl.loop(0, DIMS_PER_WORKER)
        def _(j):
            d = wid * DIMS_PER_WORKER + j
            pltpu.sync_copy(wyt_hbm.at[d], col_v)

            @pl.loop(0, n, step=16)
            def _(i):
                ids = idx_v[pl.ds(i, 16)]
                row_v[pl.ds(i, 16)] = plsc.load_gather(col_v, [ids])

            pltpu.sync_copy(row_v.at[pl.ds(0, nx)], et_hbm.at[d])
            pltpu.sync_copy(row_v.at[pl.ds(nx, ny)], ot_hbm.at[d])

    return gather_kernel(wyt, x, y, byf)


def _gru_t_body(et_ref, ht_ref, wiht_ref, whht_ref, bi_ref, bh_ref, o_ref):
    et = et_ref[...]
    ht = ht_ref[...]
    f32 = jnp.float32
    dn = (((0,), (0,)), ((), ()))
    gi = lax.dot_general(wiht_ref[...], et, dn,
                         preferred_element_type=f32) + bi_ref[...]
    gh = lax.dot_general(whht_ref[...], ht, dn,
                         preferred_element_type=f32) + bh_ref[...]
    i_r, i_z, i_n = gi[0:HID], gi[HID:2 * HID], gi[2 * HID:3 * HID]
    h_r, h_z, h_n = gh[0:HID], gh[HID:2 * HID], gh[2 * HID:3 * HID]
    r = jax.nn.sigmoid(i_r + h_r)
    z = jax.nn.sigmoid(i_z + h_z)
    nn = jnp.tanh(i_n + r * h_n)
    o_ref[...] = ((1.0 - z) * nn + z * ht).astype(jnp.bfloat16)


def _score_t_body(xht_ref, ot_ref, b_ref, r_ref):
    x = xht_ref[...]
    o = ot_ref[...].astype(jnp.bfloat16)
    acc = lax.dot_general(x, o, (((0,), (0,)), ((), ())),
                          preferred_element_type=jnp.float32)
    r_ref[...] = acc + b_ref[...]


def kernel(X, H, Y, Wy, By, weight_ih, weight_hh, bias_ih, bias_hh):
    batch = X.shape[0]
    ny = Y.shape[0]
    X = X.astype(jnp.int32)
    Y = Y.astype(jnp.int32)

    wyt = Wy.T
    ht0 = H[0].T
    wiht = weight_ih.T
    whht = weight_hh.T
    bi = bias_ih.reshape(3 * HID, 1)
    bh = bias_hh.reshape(3 * HID, 1)

    ET, OT, b1 = _sc_gather_cols(wyt, X, Y, By.reshape(-1))
    b = b1.reshape(1, ny)

    XhT = pl.pallas_call(
        _gru_t_body,
        out_shape=jax.ShapeDtypeStruct((HID, batch), jnp.bfloat16),
    )(ET, ht0, wiht, whht, bi, bh)

    bi_rows = 512
    R = pl.pallas_call(
        _score_t_body,
        grid=(batch // bi_rows,),
        in_specs=[
            pl.BlockSpec((HID, bi_rows), lambda i: (0, i)),
            pl.BlockSpec((HID, ny), lambda i: (0, 0)),
            pl.BlockSpec((1, ny), lambda i: (0, 0)),
        ],
        out_specs=pl.BlockSpec((bi_rows, ny), lambda i: (i, 0)),
        out_shape=jax.ShapeDtypeStruct((batch, ny), jnp.float32),
        compiler_params=pltpu.CompilerParams(
            dimension_semantics=("arbitrary",)),
    )(XhT, OT, b)
    return R


# speedup vs baseline: 1.6672x; 1.0230x over previous
import dataclasses
import functools

import jax
import jax.numpy as jnp
from jax import lax
from jax.experimental import pallas as pl
from jax.experimental.pallas import tpu as pltpu
from jax.experimental.pallas import tpu_sc as plsc

HID = 64
NUM_SC_CORES = 2
NUM_SC_SUBCORES = 16
NUM_WORKERS = NUM_SC_CORES * NUM_SC_SUBCORES
DIMS_PER_WORKER = HID // NUM_WORKERS


def _sc_gather_cols(wyt, x, y, byf):
    v = wyt.shape[1]
    nx = x.shape[0]
    ny = y.shape[0]
    n = nx + ny
    mesh = plsc.VectorSubcoreMesh(core_axis_name="c", subcore_axis_name="s")
    cp = pltpu.CompilerParams()
    if "needs_layout_passes" in pltpu.CompilerParams.__dataclass_fields__:
        cp = dataclasses.replace(cp, needs_layout_passes=False)

    @functools.partial(
        pl.kernel,
        mesh=mesh,
        compiler_params=cp,
        out_type=(
            jax.ShapeDtypeStruct((HID, nx), jnp.float32),
            jax.ShapeDtypeStruct((HID, ny), jnp.float32),
            jax.ShapeDtypeStruct((ny,), jnp.float32),
        ),
        scratch_types=[
            pltpu.VMEM((v,), jnp.float32),
            pltpu.VMEM((n,), jnp.int32),
            pltpu.VMEM((n,), jnp.float32),
            pltpu.VMEM((ny,), jnp.float32),
            pltpu.SemaphoreType.DMA,
        ],
    )
    def gather_kernel(wyt_hbm, x_hbm, y_hbm, byf_hbm, et_hbm, ot_hbm, b_hbm,
                      col_v, idx_v, row_v, b_v, sem):
        wid = lax.axis_index("s") * NUM_SC_CORES + lax.axis_index("c")
        pltpu.sync_copy(x_hbm, idx_v.at[pl.ds(0, nx)])
        pltpu.sync_copy(y_hbm, idx_v.at[pl.ds(nx, ny)])

        @pl.when(wid == NUM_WORKERS - 1)
        def _():
            pltpu.async_copy(byf_hbm.at[idx_v.at[pl.ds(nx, ny)]], b_v,
                             sem).wait()
            pltpu.sync_copy(b_v, b_hbm)

        @pl.loop(0, DIMS_PER_WORKER)
        def _(j):
            d = wid * DIMS_PER_WORKER + j
            pltpu.sync_copy(wyt_hbm.at[d], col_v)

            @pl.loop(0, n, step=128)
            def _(i):
                for u in range(8):
                    ids = idx_v[pl.ds(i + u * 16, 16)]
                    row_v[pl.ds(i + u * 16, 16)] = plsc.load_gather(
                        col_v, [ids])

            pltpu.sync_copy(row_v.at[pl.ds(0, nx)], et_hbm.at[d])
            pltpu.sync_copy(row_v.at[pl.ds(nx, ny)], ot_hbm.at[d])

    return gather_kernel(wyt, x, y, byf)


def _gru_t_body(et_ref, ht_ref, wiht_ref, whht_ref, bi_ref, bh_ref, o_ref):
    et = et_ref[...]
    ht = ht_ref[...]
    f32 = jnp.float32
    dn = (((0,), (0,)), ((), ()))
    gi = lax.dot_general(wiht_ref[...], et, dn,
                         preferred_element_type=f32) + bi_ref[...]
    gh = lax.dot_general(whht_ref[...], ht, dn,
                         preferred_element_type=f32) + bh_ref[...]
    i_r, i_z, i_n = gi[0:HID], gi[HID:2 * HID], gi[2 * HID:3 * HID]
    h_r, h_z, h_n = gh[0:HID], gh[HID:2 * HID], gh[2 * HID:3 * HID]
    r = jax.nn.sigmoid(i_r + h_r)
    z = jax.nn.sigmoid(i_z + h_z)
    nn = jnp.tanh(i_n + r * h_n)
    o_ref[...] = ((1.0 - z) * nn + z * ht).astype(jnp.bfloat16)


def _score_t_body(xht_ref, ot_ref, b_ref, r_ref):
    x = xht_ref[...]
    o = ot_ref[...].astype(jnp.bfloat16)
    acc = lax.dot_general(x, o, (((0,), (0,)), ((), ())),
                          preferred_element_type=jnp.float32)
    r_ref[...] = acc + b_ref[...]


def kernel(X, H, Y, Wy, By, weight_ih, weight_hh, bias_ih, bias_hh):
    batch = X.shape[0]
    ny = Y.shape[0]
    X = X.astype(jnp.int32)
    Y = Y.astype(jnp.int32)

    wyt = Wy.T
    ht0 = H[0].T
    wiht = weight_ih.T
    whht = weight_hh.T
    bi = bias_ih.reshape(3 * HID, 1)
    bh = bias_hh.reshape(3 * HID, 1)

    ET, OT, b1 = _sc_gather_cols(wyt, X, Y, By.reshape(-1))
    b = b1.reshape(1, ny)

    XhT = pl.pallas_call(
        _gru_t_body,
        out_shape=jax.ShapeDtypeStruct((HID, batch), jnp.bfloat16),
    )(ET, ht0, wiht, whht, bi, bh)

    bi_rows = 512
    R = pl.pallas_call(
        _score_t_body,
        grid=(batch // bi_rows,),
        in_specs=[
            pl.BlockSpec((HID, bi_rows), lambda i: (0, i)),
            pl.BlockSpec((HID, ny), lambda i: (0, 0)),
            pl.BlockSpec((1, ny), lambda i: (0, 0)),
        ],
        out_specs=pl.BlockSpec((bi_rows, ny), lambda i: (i, 0)),
        out_shape=jax.ShapeDtypeStruct((batch, ny), jnp.float32),
        compiler_params=pltpu.CompilerParams(
            dimension_semantics=("arbitrary",)),
    )(XhT, OT, b)
    return R


# speedup vs baseline: 1.7102x; 1.0258x over previous
import dataclasses
import functools

import jax
import jax.numpy as jnp
from jax import lax
from jax.experimental import pallas as pl
from jax.experimental.pallas import tpu as pltpu
from jax.experimental.pallas import tpu_sc as plsc

HID = 64
NUM_SC_CORES = 2
NUM_SC_SUBCORES = 16
NUM_WORKERS = NUM_SC_CORES * NUM_SC_SUBCORES
DIMS_PER_WORKER = HID // NUM_WORKERS


def _sc_gather_cols(wyt, x, y, byf):
    v = wyt.shape[1]
    nx = x.shape[0]
    ny = y.shape[0]
    n = nx + ny
    mesh = plsc.VectorSubcoreMesh(core_axis_name="c", subcore_axis_name="s")
    cp = pltpu.CompilerParams()
    if "needs_layout_passes" in pltpu.CompilerParams.__dataclass_fields__:
        cp = dataclasses.replace(cp, needs_layout_passes=False)

    @functools.partial(
        pl.kernel,
        mesh=mesh,
        compiler_params=cp,
        out_type=(
            jax.ShapeDtypeStruct((HID, nx), jnp.float32),
            jax.ShapeDtypeStruct((HID, ny), jnp.float32),
            jax.ShapeDtypeStruct((ny,), jnp.float32),
        ),
        scratch_types=[
            pltpu.VMEM((v,), jnp.float32),
            pltpu.VMEM((n,), jnp.int32),
            pltpu.VMEM((n,), jnp.float32),
            pltpu.VMEM((ny,), jnp.float32),
            pltpu.SemaphoreType.DMA,
        ],
    )
    def gather_kernel(wyt_hbm, x_hbm, y_hbm, byf_hbm, et_hbm, ot_hbm, b_hbm,
                      col_v, idx_v, row_v, b_v, sem):
        wid = lax.axis_index("s") * NUM_SC_CORES + lax.axis_index("c")
        pltpu.sync_copy(x_hbm, idx_v.at[pl.ds(0, nx)])
        pltpu.sync_copy(y_hbm, idx_v.at[pl.ds(nx, ny)])

        @pl.when(wid == NUM_WORKERS - 1)
        def _():
            pltpu.async_copy(byf_hbm.at[idx_v.at[pl.ds(nx, ny)]], b_v,
                             sem).wait()
            pltpu.sync_copy(b_v, b_hbm)

        @pl.loop(0, DIMS_PER_WORKER)
        def _(j):
            d = wid * DIMS_PER_WORKER + j
            pltpu.sync_copy(wyt_hbm.at[d], col_v)

            @pl.loop(0, n, step=128)
            def _(i):
                for u in range(8):
                    ids = idx_v[pl.ds(i + u * 16, 16)]
                    row_v[pl.ds(i + u * 16, 16)] = plsc.load_gather(
                        col_v, [ids])

            pltpu.sync_copy(row_v.at[pl.ds(0, nx)], et_hbm.at[d])
            pltpu.sync_copy(row_v.at[pl.ds(nx, ny)], ot_hbm.at[d])

    return gather_kernel(wyt, x, y, byf)


BI_ROWS = 512


def _fused_t_body(et_ref, ht_ref, wiht_ref, whht_ref, bi_ref, bh_ref,
                  ot_ref, b_ref, r_ref, xht_s):
    i = pl.program_id(0)

    @pl.when(i == 0)
    def _():
        et = et_ref[...]
        ht = ht_ref[...]
        f32 = jnp.float32
        dn = (((0,), (0,)), ((), ()))
        gi = lax.dot_general(wiht_ref[...], et, dn,
                             preferred_element_type=f32) + bi_ref[...]
        gh = lax.dot_general(whht_ref[...], ht, dn,
                             preferred_element_type=f32) + bh_ref[...]
        i_r, i_z, i_n = gi[0:HID], gi[HID:2 * HID], gi[2 * HID:3 * HID]
        h_r, h_z, h_n = gh[0:HID], gh[HID:2 * HID], gh[2 * HID:3 * HID]
        r = jax.nn.sigmoid(i_r + h_r)
        z = jax.nn.sigmoid(i_z + h_z)
        nn = jnp.tanh(i_n + r * h_n)
        xht_s[...] = ((1.0 - z) * nn + z * ht).astype(jnp.bfloat16)

    x = xht_s[:, pl.ds(i * BI_ROWS, BI_ROWS)]
    o = ot_ref[...].astype(jnp.bfloat16)
    acc = lax.dot_general(x, o, (((0,), (0,)), ((), ())),
                          preferred_element_type=jnp.float32)
    r_ref[...] = acc + b_ref[...]


def kernel(X, H, Y, Wy, By, weight_ih, weight_hh, bias_ih, bias_hh):
    batch = X.shape[0]
    ny = Y.shape[0]
    X = X.astype(jnp.int32)
    Y = Y.astype(jnp.int32)

    wyt = Wy.T
    ht0 = H[0].T
    wiht = weight_ih.T
    whht = weight_hh.T
    bi = bias_ih.reshape(3 * HID, 1)
    bh = bias_hh.reshape(3 * HID, 1)

    ET, OT, b1 = _sc_gather_cols(wyt, X, Y, By.reshape(-1))
    b = b1.reshape(1, ny)

    const = lambda i: (0, 0)
    R = pl.pallas_call(
        _fused_t_body,
        grid=(batch // BI_ROWS,),
        in_specs=[
            pl.BlockSpec((HID, batch), const),
            pl.BlockSpec((HID, batch), const),
            pl.BlockSpec((HID, 3 * HID), const),
            pl.BlockSpec((HID, 3 * HID), const),
            pl.BlockSpec((3 * HID, 1), const),
            pl.BlockSpec((3 * HID, 1), const),
            pl.BlockSpec((HID, ny), const),
            pl.BlockSpec((1, ny), const),
        ],
        out_specs=pl.BlockSpec((BI_ROWS, ny), lambda i: (i, 0)),
        out_shape=jax.ShapeDtypeStruct((batch, ny), jnp.float32),
        scratch_shapes=[pltpu.VMEM((HID, batch), jnp.bfloat16)],
        compiler_params=pltpu.CompilerParams(
            dimension_semantics=("arbitrary",)),
    )(ET, ht0, wiht, whht, bi, bh, OT, b)
    return R


# speedup vs baseline: 1.7513x; 1.0240x over previous
import dataclasses
import functools

import jax
import jax.numpy as jnp
from jax import lax
from jax.experimental import pallas as pl
from jax.experimental.pallas import tpu as pltpu
from jax.experimental.pallas import tpu_sc as plsc

HID = 64
NUM_SC_CORES = 2
NUM_SC_SUBCORES = 16
NUM_WORKERS = NUM_SC_CORES * NUM_SC_SUBCORES
DIMS_PER_WORKER = HID // NUM_WORKERS


def _sc_gather_cols(wyt, x, y, byf):
    v = wyt.shape[1]
    nx = x.shape[0]
    ny = y.shape[0]
    n = nx + ny
    mesh = plsc.VectorSubcoreMesh(core_axis_name="c", subcore_axis_name="s")
    cp = pltpu.CompilerParams()
    if "needs_layout_passes" in pltpu.CompilerParams.__dataclass_fields__:
        cp = dataclasses.replace(cp, needs_layout_passes=False)

    @functools.partial(
        pl.kernel,
        mesh=mesh,
        compiler_params=cp,
        out_type=(
            jax.ShapeDtypeStruct((HID, nx), jnp.float32),
            jax.ShapeDtypeStruct((HID, ny), jnp.float32),
            jax.ShapeDtypeStruct((ny,), jnp.float32),
        ),
        scratch_types=[
            pltpu.VMEM((v,), jnp.float32),
            pltpu.VMEM((n,), jnp.int32),
            pltpu.VMEM((n,), jnp.float32),
            pltpu.VMEM((ny,), jnp.float32),
            pltpu.SemaphoreType.DMA,
        ],
    )
    def gather_kernel(wyt_hbm, x_hbm, y_hbm, byf_hbm, et_hbm, ot_hbm, b_hbm,
                      col_v, idx_v, row_v, b_v, sem):
        wid = lax.axis_index("s") * NUM_SC_CORES + lax.axis_index("c")
        pltpu.sync_copy(x_hbm, idx_v.at[pl.ds(0, nx)])
        pltpu.sync_copy(y_hbm, idx_v.at[pl.ds(nx, ny)])

        nb = ny // 128

        @pl.when(wid == NUM_WORKERS - 1)
        def _():
            @pl.loop(0, nb)
            def _(k):
                pltpu.async_copy(
                    byf_hbm.at[idx_v.at[pl.ds(nx + k * 128, 128)]],
                    b_v.at[pl.ds(k * 128, 128)], sem)

        @pl.loop(0, DIMS_PER_WORKER)
        def _(j):
            d = wid * DIMS_PER_WORKER + j
            pltpu.sync_copy(wyt_hbm.at[d], col_v)

            @pl.loop(0, n, step=128)
            def _(i):
                for u in range(8):
                    ids = idx_v[pl.ds(i + u * 16, 16)]
                    row_v[pl.ds(i + u * 16, 16)] = plsc.load_gather(
                        col_v, [ids])

            pltpu.sync_copy(row_v.at[pl.ds(0, nx)], et_hbm.at[d])
            pltpu.sync_copy(row_v.at[pl.ds(nx, ny)], ot_hbm.at[d])

        @pl.when(wid == NUM_WORKERS - 1)
        def _():
            @pl.loop(0, nb)
            def _(k):
                pltpu.make_async_copy(
                    byf_hbm.at[pl.ds(0, 128)],
                    b_v.at[pl.ds(k * 128, 128)], sem).wait()

            pltpu.sync_copy(b_v, b_hbm)

    return gather_kernel(wyt, x, y, byf)


BI_ROWS = 512


def _fused_t_body(et_ref, ht_ref, wiht_ref, whht_ref, bi_ref, bh_ref,
                  ot_ref, b_ref, r_ref, xht_s):
    i = pl.program_id(0)

    @pl.when(i == 0)
    def _():
        et = et_ref[...]
        ht = ht_ref[...]
        f32 = jnp.float32
        dn = (((0,), (0,)), ((), ()))
        gi = lax.dot_general(wiht_ref[...], et, dn,
                             preferred_element_type=f32) + bi_ref[...]
        gh = lax.dot_general(whht_ref[...], ht, dn,
                             preferred_element_type=f32) + bh_ref[...]
        i_r, i_z, i_n = gi[0:HID], gi[HID:2 * HID], gi[2 * HID:3 * HID]
        h_r, h_z, h_n = gh[0:HID], gh[HID:2 * HID], gh[2 * HID:3 * HID]
        r = jax.nn.sigmoid(i_r + h_r)
        z = jax.nn.sigmoid(i_z + h_z)
        nn = jnp.tanh(i_n + r * h_n)
        xht_s[...] = ((1.0 - z) * nn + z * ht).astype(jnp.bfloat16)

    x = xht_s[:, pl.ds(i * BI_ROWS, BI_ROWS)]
    o = ot_ref[...].astype(jnp.bfloat16)
    acc = lax.dot_general(x, o, (((0,), (0,)), ((), ())),
                          preferred_element_type=jnp.float32)
    r_ref[...] = acc + b_ref[...]


def kernel(X, H, Y, Wy, By, weight_ih, weight_hh, bias_ih, bias_hh):
    batch = X.shape[0]
    ny = Y.shape[0]
    X = X.astype(jnp.int32)
    Y = Y.astype(jnp.int32)

    wyt = Wy.T
    ht0 = H[0].T
    wiht = weight_ih.T
    whht = weight_hh.T
    bi = bias_ih.reshape(3 * HID, 1)
    bh = bias_hh.reshape(3 * HID, 1)

    ET, OT, b1 = _sc_gather_cols(wyt, X, Y, By.reshape(-1))
    b = b1.reshape(1, ny)

    const = lambda i: (0, 0)
    R = pl.pallas_call(
        _fused_t_body,
        grid=(batch // BI_ROWS,),
        in_specs=[
            pl.BlockSpec((HID, batch), const),
            pl.BlockSpec((HID, batch), const),
            pl.BlockSpec((HID, 3 * HID), const),
            pl.BlockSpec((HID, 3 * HID), const),
            pl.BlockSpec((3 * HID, 1), const),
            pl.BlockSpec((3 * HID, 1), const),
            pl.BlockSpec((HID, ny), const),
            pl.BlockSpec((1, ny), const),
        ],
        out_specs=pl.BlockSpec((BI_ROWS, ny), lambda i: (i, 0)),
        out_shape=jax.ShapeDtypeStruct((batch, ny), jnp.float32),
        scratch_shapes=[pltpu.VMEM((HID, batch), jnp.bfloat16)],
        compiler_params=pltpu.CompilerParams(
            dimension_semantics=("arbitrary",)),
    )(ET, ht0, wiht, whht, bi, bh, OT, b)
    return R
